# Initial kernel scaffold; baseline (speedup 1.0000x reference)
import jax
import jax.numpy as jnp
from jax.experimental import pallas as pl


def kernel(x, edge_index, edge_attr, eps0, W1_0, b1_0, bn_g0, bn_b0, W2_0, b2_0, eps1, W1_1, b1_1, bn_g1, bn_b1, W2_1, b2_1, lin_W, lin_b):
    raise NotImplementedError("write your pallas kernel here")


# speedup vs baseline: 12.2850x; 12.2850x over previous
import functools

import jax
import jax.numpy as jnp
from jax import lax
from jax.experimental import pallas as pl
from jax.experimental.pallas import tpu as pltpu
from jax.experimental.pallas import tpu_sc as plsc

N = 10000
E = 320000
D = 128
G2 = 256
BN_EPS = 1e-5

NC = 2
NS = 16
NW = NC * NS
PER_W = E // NW
CHUNK = 125
NCH = PER_W // CHUNK
NHALF = 2
NCHH = NCH // NHALF
NROW = 624
NTAIL = N - NS * NROW

BLK = 2000
NBLK = N // BLK


def _sc_segment_sum(h, src2d, dst2d, zrows):
  mesh = plsc.VectorSubcoreMesh(core_axis_name="c", subcore_axis_name="s")

  @functools.partial(
      pl.kernel,
      out_type=jax.ShapeDtypeStruct((NC, N, D), jnp.float32),
      mesh=mesh,
      scratch_types=[
          pltpu.VMEM((NCHH, CHUNK), jnp.int32),
          pltpu.VMEM((NCHH, CHUNK), jnp.int32),
          pltpu.VMEM((2, CHUNK, D), jnp.float32),
          pltpu.VMEM_SHARED((N, D), jnp.float32),
          pltpu.SemaphoreType.DMA,
          pltpu.SemaphoreType.DMA,
      ],
  )
  def agg_kernel(h_hbm, src_hbm, dst_hbm, z_hbm, out_hbm,
                 idx_s, idx_d, rows, acc, sem0, sem1):
    c = lax.axis_index("c")
    s = lax.axis_index("s")
    gw = c * NS + s
    pltpu.sync_copy(z_hbm.at[pl.ds(0, NROW)], acc.at[pl.ds(s * NROW, NROW)])

    @pl.when(s == NS - 1)
    def _():
      pltpu.sync_copy(z_hbm.at[pl.ds(0, NTAIL)],
                      acc.at[pl.ds(NS * NROW, NTAIL)])

    for half in range(NHALF):
      base = gw * NCH + half * NCHH
      pltpu.sync_copy(src_hbm.at[pl.ds(base, NCHH)], idx_s)
      pltpu.sync_copy(dst_hbm.at[pl.ds(base, NCHH)], idx_d)
      if half == 0:
        plsc.subcore_barrier()

      pltpu.async_copy(h_hbm.at[idx_s.at[0]], rows.at[0], sem0)

      def body(jj, carry):
        j0 = 2 * jj
        pltpu.async_copy(h_hbm.at[idx_s.at[j0 + 1]], rows.at[1], sem1)
        pltpu.make_async_copy(h_hbm.at[idx_s.at[j0]], rows.at[0], sem0).wait()
        pltpu.sync_copy(rows.at[0], acc.at[idx_d.at[j0]], add=True)

        @pl.when(jj + 1 < NCHH // 2)
        def _():
          pltpu.async_copy(h_hbm.at[idx_s.at[j0 + 2]], rows.at[0], sem0)

        pltpu.make_async_copy(h_hbm.at[idx_s.at[j0 + 1]], rows.at[1],
                              sem1).wait()
        pltpu.sync_copy(rows.at[1], acc.at[idx_d.at[j0 + 1]], add=True)
        return carry

      lax.fori_loop(0, NCHH // 2, body, 0)

    plsc.subcore_barrier()
    pltpu.sync_copy(acc.at[pl.ds(s * NROW, NROW)],
                    out_hbm.at[c, pl.ds(s * NROW, NROW)])

    @pl.when(s == NS - 1)
    def _():
      pltpu.sync_copy(acc.at[pl.ds(NS * NROW, NTAIL)],
                      out_hbm.at[c, pl.ds(NS * NROW, NTAIL)])

  return agg_kernel(h, src2d, dst2d, zrows)


def _dot(a, b):
  return lax.dot_general(a, b, (((1,), (0,)), ((), ())),
                         preferred_element_type=jnp.float32)


def _mlp1_body(scale_ref, x_ref, agg_ref, w1_ref, b1_ref, w2_ref, b2_ref,
               o_ref):
  z = x_ref[...] * scale_ref[...] + agg_ref[0] + agg_ref[1]
  u = jnp.tanh(_dot(z, w1_ref[...]) + b1_ref[...])
  o_ref[...] = jnp.tanh(_dot(u, w2_ref[...]) + b2_ref[...])


def _mlp2_body(scale_ref, h_ref, agg_ref, w1_ref, b1_ref, w2_ref, b2_ref,
               lw_ref, lb_ref, o_ref):
  z = h_ref[...] * scale_ref[...] + agg_ref[0] + agg_ref[1]
  u = jnp.tanh(_dot(z, w1_ref[...]) + b1_ref[...])
  v = jnp.tanh(_dot(u, w2_ref[...]) + b2_ref[...])
  o_ref[...] = jnp.tanh(_dot(v, lw_ref[...]) + lb_ref[...])


def _full(shape):
  return pl.BlockSpec(shape, lambda i: tuple(0 for _ in shape))


def _tc_layer1(x, agg, scale, w1, b1, w2, b2):
  return pl.pallas_call(
      _mlp1_body,
      grid=(NBLK,),
      in_specs=[
          _full((1, D)),
          pl.BlockSpec((BLK, D), lambda i: (i, 0)),
          pl.BlockSpec((NC, BLK, D), lambda i: (0, i, 0)),
          _full((D, G2)),
          _full((1, G2)),
          _full((G2, D)),
          _full((1, D)),
      ],
      out_specs=pl.BlockSpec((BLK, D), lambda i: (i, 0)),
      out_shape=jax.ShapeDtypeStruct((N, D), jnp.float32),
  )(scale, x, agg, w1, b1, w2, b2)


def _tc_layer2(h, agg, scale, w1, b1, w2, b2, lw, lb):
  return pl.pallas_call(
      _mlp2_body,
      grid=(NBLK,),
      in_specs=[
          _full((1, D)),
          pl.BlockSpec((BLK, D), lambda i: (i, 0)),
          pl.BlockSpec((NC, BLK, D), lambda i: (0, i, 0)),
          _full((D, G2)),
          _full((1, G2)),
          _full((G2, D)),
          _full((1, D)),
          _full((D, D)),
          _full((1, D)),
      ],
      out_specs=pl.BlockSpec((BLK, D), lambda i: (i, 0)),
      out_shape=jax.ShapeDtypeStruct((N, D), jnp.float32),
  )(scale, h, agg, w1, b1, w2, b2, lw, lb)


def kernel(x, edge_index, edge_attr, eps0, W1_0, b1_0, bn_g0, bn_b0, W2_0,
           b2_0, eps1, W1_1, b1_1, bn_g1, bn_b1, W2_1, b2_1, lin_W, lin_b):
  src2d = edge_index[0].reshape(NW * NCH, CHUNK)
  dst2d = edge_index[1].reshape(NW * NCH, CHUNK)
  zrows = jnp.zeros((NROW, D), jnp.float32)

  bnscale = 1.0 / jnp.sqrt(jnp.float32(1.0 + BN_EPS))
  g0 = bn_g0 * bnscale
  w1f0 = W1_0 * g0[None, :]
  b1f0 = (b1_0 * g0 + bn_b0).reshape(1, G2)
  g1 = bn_g1 * bnscale
  w1f1 = W1_1 * g1[None, :]
  b1f1 = (b1_1 * g1 + bn_b1).reshape(1, G2)
  scale0 = jnp.full((1, D), 1.0, jnp.float32) * (1.0 + eps0)
  scale1 = jnp.full((1, D), 1.0, jnp.float32) * (1.0 + eps1)

  agg0 = _sc_segment_sum(x, src2d, dst2d, zrows)
  h1 = _tc_layer1(x, agg0, scale0, w1f0, b1f0, W2_0, b2_0.reshape(1, D))
  agg1 = _sc_segment_sum(h1, src2d, dst2d, zrows)
  out = _tc_layer2(h1, agg1, scale1, w1f1, b1f1, W2_1, b2_1.reshape(1, D),
                   lin_W, lin_b.reshape(1, D))
  return out


# speedup vs baseline: 12.3652x; 1.0065x over previous
import functools

import jax
import jax.numpy as jnp
from jax import lax
from jax.experimental import pallas as pl
from jax.experimental.pallas import tpu as pltpu
from jax.experimental.pallas import tpu_sc as plsc

N = 10000
E = 320000
D = 128
G2 = 256
BN_EPS = 1e-5

NC = 2
NS = 16
NW = NC * NS
PER_W = E // NW
CHUNK = 125
NCH = PER_W // CHUNK
NHALF = 2
NCHH = NCH // NHALF
NROW = 624
NTAIL = N - NS * NROW

BLK = 2000
NBLK = N // BLK


def _sc_segment_sum(h, src2d, dst2d, zrows):
  mesh = plsc.VectorSubcoreMesh(core_axis_name="c", subcore_axis_name="s")

  @functools.partial(
      pl.kernel,
      out_type=jax.ShapeDtypeStruct((NC, N, D), jnp.float32),
      mesh=mesh,
      scratch_types=[
          pltpu.VMEM((NCHH, CHUNK), jnp.int32),
          pltpu.VMEM((NCHH, CHUNK), jnp.int32),
          pltpu.VMEM((2, CHUNK, D), jnp.float32),
          pltpu.VMEM_SHARED((N, D), jnp.float32),
          pltpu.SemaphoreType.DMA,
          pltpu.SemaphoreType.DMA,
      ],
  )
  def agg_kernel(h_hbm, src_hbm, dst_hbm, z_hbm, out_hbm,
                 idx_s, idx_d, rows, acc, sem0, sem1):
    c = lax.axis_index("c")
    s = lax.axis_index("s")
    gw = c * NS + s
    pltpu.sync_copy(z_hbm.at[pl.ds(0, NROW)], acc.at[pl.ds(s * NROW, NROW)])

    @pl.when(s == NS - 1)
    def _():
      pltpu.sync_copy(z_hbm.at[pl.ds(0, NTAIL)],
                      acc.at[pl.ds(NS * NROW, NTAIL)])

    for half in range(NHALF):
      base = gw * NCH + half * NCHH
      pltpu.sync_copy(src_hbm.at[pl.ds(base, NCHH)], idx_s)
      pltpu.sync_copy(dst_hbm.at[pl.ds(base, NCHH)], idx_d)
      if half == 0:
        plsc.subcore_barrier()

      pltpu.async_copy(h_hbm.at[idx_s.at[0]], rows.at[0], sem0)

      def body(jj, carry):
        j0 = 2 * jj
        pltpu.async_copy(h_hbm.at[idx_s.at[j0 + 1]], rows.at[1], sem1)
        pltpu.make_async_copy(h_hbm.at[idx_s.at[j0]], rows.at[0], sem0).wait()
        pltpu.sync_copy(rows.at[0], acc.at[idx_d.at[j0]], add=True)

        @pl.when(jj + 1 < NCHH // 2)
        def _():
          pltpu.async_copy(h_hbm.at[idx_s.at[j0 + 2]], rows.at[0], sem0)

        pltpu.make_async_copy(h_hbm.at[idx_s.at[j0 + 1]], rows.at[1],
                              sem1).wait()
        pltpu.sync_copy(rows.at[1], acc.at[idx_d.at[j0 + 1]], add=True)
        return carry

      lax.fori_loop(0, NCHH // 2, body, 0)

    plsc.subcore_barrier()
    pltpu.sync_copy(acc.at[pl.ds(s * NROW, NROW)],
                    out_hbm.at[c, pl.ds(s * NROW, NROW)])

    @pl.when(s == NS - 1)
    def _():
      pltpu.sync_copy(acc.at[pl.ds(NS * NROW, NTAIL)],
                      out_hbm.at[c, pl.ds(NS * NROW, NTAIL)])

  return agg_kernel(h, src2d, dst2d, zrows)


def _dot(a, b):
  return lax.dot_general(a, b, (((1,), (0,)), ((), ())),
                         preferred_element_type=jnp.float32)


_BN_SCALE = (1.0 + BN_EPS) ** -0.5


def _gin_mlp(h, agg_ref, eps_ref, w1_ref, b1_ref, g_ref, bb_ref, w2_ref,
             b2_ref):
  z = h * (1.0 + eps_ref[0, 0]) + agg_ref[0] + agg_ref[1]
  gs = g_ref[...] * _BN_SCALE
  u = jnp.tanh(_dot(z, w1_ref[...]) * gs + b1_ref[...] * gs + bb_ref[...])
  return jnp.tanh(_dot(u, w2_ref[...]) + b2_ref[...])


def _mlp1_body(eps_ref, x_ref, agg_ref, w1_ref, b1_ref, g_ref, bb_ref,
               w2_ref, b2_ref, o_ref):
  o_ref[...] = _gin_mlp(x_ref[...], agg_ref, eps_ref, w1_ref, b1_ref, g_ref,
                        bb_ref, w2_ref, b2_ref)


def _mlp2_body(eps_ref, h_ref, agg_ref, w1_ref, b1_ref, g_ref, bb_ref,
               w2_ref, b2_ref, lw_ref, lb_ref, o_ref):
  v = _gin_mlp(h_ref[...], agg_ref, eps_ref, w1_ref, b1_ref, g_ref, bb_ref,
               w2_ref, b2_ref)
  o_ref[...] = jnp.tanh(_dot(v, lw_ref[...]) + lb_ref[...])


def _full(shape):
  return pl.BlockSpec(shape, lambda i: tuple(0 for _ in shape))


def _tc_layer1(x, agg, eps, w1, b1, g, bb, w2, b2):
  return pl.pallas_call(
      _mlp1_body,
      grid=(NBLK,),
      in_specs=[
          _full((1, 1)),
          pl.BlockSpec((BLK, D), lambda i: (i, 0)),
          pl.BlockSpec((NC, BLK, D), lambda i: (0, i, 0)),
          _full((D, G2)),
          _full((1, G2)),
          _full((1, G2)),
          _full((1, G2)),
          _full((G2, D)),
          _full((1, D)),
      ],
      out_specs=pl.BlockSpec((BLK, D), lambda i: (i, 0)),
      out_shape=jax.ShapeDtypeStruct((N, D), jnp.float32),
  )(eps, x, agg, w1, b1, g, bb, w2, b2)


def _tc_layer2(h, agg, eps, w1, b1, g, bb, w2, b2, lw, lb):
  return pl.pallas_call(
      _mlp2_body,
      grid=(NBLK,),
      in_specs=[
          _full((1, 1)),
          pl.BlockSpec((BLK, D), lambda i: (i, 0)),
          pl.BlockSpec((NC, BLK, D), lambda i: (0, i, 0)),
          _full((D, G2)),
          _full((1, G2)),
          _full((1, G2)),
          _full((1, G2)),
          _full((G2, D)),
          _full((1, D)),
          _full((D, D)),
          _full((1, D)),
      ],
      out_specs=pl.BlockSpec((BLK, D), lambda i: (i, 0)),
      out_shape=jax.ShapeDtypeStruct((N, D), jnp.float32),
  )(eps, h, agg, w1, b1, g, bb, w2, b2, lw, lb)


def kernel(x, edge_index, edge_attr, eps0, W1_0, b1_0, bn_g0, bn_b0, W2_0,
           b2_0, eps1, W1_1, b1_1, bn_g1, bn_b1, W2_1, b2_1, lin_W, lin_b):
  src2d = edge_index[0].reshape(NW * NCH, CHUNK)
  dst2d = edge_index[1].reshape(NW * NCH, CHUNK)
  zrows = jnp.zeros((NROW, D), jnp.float32)

  agg0 = _sc_segment_sum(x, src2d, dst2d, zrows)
  h1 = _tc_layer1(x, agg0, eps0.reshape(1, 1), W1_0, b1_0.reshape(1, G2),
                  bn_g0.reshape(1, G2), bn_b0.reshape(1, G2), W2_0,
                  b2_0.reshape(1, D))
  agg1 = _sc_segment_sum(h1, src2d, dst2d, zrows)
  out = _tc_layer2(h1, agg1, eps1.reshape(1, 1), W1_1, b1_1.reshape(1, G2),
                   bn_g1.reshape(1, G2), bn_b1.reshape(1, G2), W2_1,
                   b2_1.reshape(1, D), lin_W, lin_b.reshape(1, D))
  return out


# speedup vs baseline: 12.4711x; 1.0086x over previous
import functools

import jax
import jax.numpy as jnp
from jax import lax
from jax.experimental import pallas as pl
from jax.experimental.pallas import tpu as pltpu
from jax.experimental.pallas import tpu_sc as plsc

N = 10000
E = 320000
D = 128
G2 = 256
BN_EPS = 1e-5

NC = 2
NS = 16
NW = NC * NS
PER_W = E // NW
CHUNK = 125
NCH = PER_W // CHUNK
NHALF = 2
NCHH = NCH // NHALF
NROW = 624
NTAIL = N - NS * NROW

BLK = 2000
NBLK = N // BLK


def _sc_segment_sum(h, src2d, dst2d, zrows):
  mesh = plsc.VectorSubcoreMesh(core_axis_name="c", subcore_axis_name="s")

  @functools.partial(
      pl.kernel,
      out_type=jax.ShapeDtypeStruct((NC, N, D), jnp.float32),
      mesh=mesh,
      scratch_types=[
          pltpu.VMEM((NCHH, CHUNK), jnp.int32),
          pltpu.VMEM((NCHH, CHUNK), jnp.int32),
          pltpu.VMEM((2, CHUNK, D), jnp.float32),
          pltpu.VMEM_SHARED((N, D), jnp.float32),
          pltpu.SemaphoreType.DMA,
          pltpu.SemaphoreType.DMA,
          pltpu.SemaphoreType.DMA,
      ],
  )
  def agg_kernel(h_hbm, src_hbm, dst_hbm, z_hbm, out_hbm,
                 idx_s, idx_d, rows, acc, sem0, sem1, semz):
    c = lax.axis_index("c")
    s = lax.axis_index("s")
    gw = c * NS + s
    pltpu.async_copy(z_hbm.at[pl.ds(0, NROW)],
                     acc.at[pl.ds(s * NROW, NROW)], semz)

    @pl.when(s == NS - 1)
    def _():
      pltpu.sync_copy(z_hbm.at[pl.ds(0, NTAIL)],
                      acc.at[pl.ds(NS * NROW, NTAIL)])

    for half in range(NHALF):
      base = gw * NCH + half * NCHH
      pltpu.sync_copy(src_hbm.at[pl.ds(base, NCHH)], idx_s)
      pltpu.sync_copy(dst_hbm.at[pl.ds(base, NCHH)], idx_d)
      if half == 0:
        pltpu.make_async_copy(z_hbm.at[pl.ds(0, NROW)],
                              acc.at[pl.ds(s * NROW, NROW)], semz).wait()
        plsc.subcore_barrier()

      pltpu.async_copy(h_hbm.at[idx_s.at[0]], rows.at[0], sem0)

      def body(jj, carry):
        j0 = 2 * jj
        pltpu.async_copy(h_hbm.at[idx_s.at[j0 + 1]], rows.at[1], sem1)
        pltpu.make_async_copy(h_hbm.at[idx_s.at[j0]], rows.at[0], sem0).wait()
        pltpu.sync_copy(rows.at[0], acc.at[idx_d.at[j0]], add=True)

        @pl.when(jj + 1 < NCHH // 2)
        def _():
          pltpu.async_copy(h_hbm.at[idx_s.at[j0 + 2]], rows.at[0], sem0)

        pltpu.make_async_copy(h_hbm.at[idx_s.at[j0 + 1]], rows.at[1],
                              sem1).wait()
        pltpu.sync_copy(rows.at[1], acc.at[idx_d.at[j0 + 1]], add=True)
        return carry

      lax.fori_loop(0, NCHH // 2, body, 0)

    plsc.subcore_barrier()
    pltpu.sync_copy(acc.at[pl.ds(s * NROW, NROW)],
                    out_hbm.at[c, pl.ds(s * NROW, NROW)])

    @pl.when(s == NS - 1)
    def _():
      pltpu.sync_copy(acc.at[pl.ds(NS * NROW, NTAIL)],
                      out_hbm.at[c, pl.ds(NS * NROW, NTAIL)])

  return agg_kernel(h, src2d, dst2d, zrows)


def _dot(a, b):
  return lax.dot_general(a.astype(jnp.bfloat16), b.astype(jnp.bfloat16),
                         (((1,), (0,)), ((), ())),
                         preferred_element_type=jnp.float32)


_BN_SCALE = (1.0 + BN_EPS) ** -0.5


def _gin_mlp(h, agg_ref, eps_ref, w1_ref, b1_ref, g_ref, bb_ref, w2_ref,
             b2_ref):
  z = h * (1.0 + eps_ref[0, 0]) + agg_ref[0] + agg_ref[1]
  gs = g_ref[...] * _BN_SCALE
  u = jnp.tanh(_dot(z, w1_ref[...]) * gs + b1_ref[...] * gs + bb_ref[...])
  return jnp.tanh(_dot(u, w2_ref[...]) + b2_ref[...])


def _mlp1_body(eps_ref, x_ref, agg_ref, w1_ref, b1_ref, g_ref, bb_ref,
               w2_ref, b2_ref, o_ref):
  o_ref[...] = _gin_mlp(x_ref[...], agg_ref, eps_ref, w1_ref, b1_ref, g_ref,
                        bb_ref, w2_ref, b2_ref)


def _mlp2_body(eps_ref, h_ref, agg_ref, w1_ref, b1_ref, g_ref, bb_ref,
               w2_ref, b2_ref, lw_ref, lb_ref, o_ref):
  v = _gin_mlp(h_ref[...], agg_ref, eps_ref, w1_ref, b1_ref, g_ref, bb_ref,
               w2_ref, b2_ref)
  o_ref[...] = jnp.tanh(_dot(v, lw_ref[...]) + lb_ref[...])


def _full(shape):
  return pl.BlockSpec(shape, lambda i: tuple(0 for _ in shape))


def _tc_layer1(x, agg, eps, w1, b1, g, bb, w2, b2):
  return pl.pallas_call(
      _mlp1_body,
      grid=(NBLK,),
      in_specs=[
          _full((1, 1)),
          pl.BlockSpec((BLK, D), lambda i: (i, 0)),
          pl.BlockSpec((NC, BLK, D), lambda i: (0, i, 0)),
          _full((D, G2)),
          _full((1, G2)),
          _full((1, G2)),
          _full((1, G2)),
          _full((G2, D)),
          _full((1, D)),
      ],
      out_specs=pl.BlockSpec((BLK, D), lambda i: (i, 0)),
      out_shape=jax.ShapeDtypeStruct((N, D), jnp.float32),
  )(eps, x, agg, w1, b1, g, bb, w2, b2)


def _tc_layer2(h, agg, eps, w1, b1, g, bb, w2, b2, lw, lb):
  return pl.pallas_call(
      _mlp2_body,
      grid=(NBLK,),
      in_specs=[
          _full((1, 1)),
          pl.BlockSpec((BLK, D), lambda i: (i, 0)),
          pl.BlockSpec((NC, BLK, D), lambda i: (0, i, 0)),
          _full((D, G2)),
          _full((1, G2)),
          _full((1, G2)),
          _full((1, G2)),
          _full((G2, D)),
          _full((1, D)),
          _full((D, D)),
          _full((1, D)),
      ],
      out_specs=pl.BlockSpec((BLK, D), lambda i: (i, 0)),
      out_shape=jax.ShapeDtypeStruct((N, D), jnp.float32),
  )(eps, h, agg, w1, b1, g, bb, w2, b2, lw, lb)


def kernel(x, edge_index, edge_attr, eps0, W1_0, b1_0, bn_g0, bn_b0, W2_0,
           b2_0, eps1, W1_1, b1_1, bn_g1, bn_b1, W2_1, b2_1, lin_W, lin_b):
  src2d = edge_index[0].reshape(NW * NCH, CHUNK)
  dst2d = edge_index[1].reshape(NW * NCH, CHUNK)
  zrows = jnp.zeros((NROW, D), jnp.float32)

  agg0 = _sc_segment_sum(x, src2d, dst2d, zrows)
  h1 = _tc_layer1(x, agg0, eps0.reshape(1, 1), W1_0, b1_0.reshape(1, G2),
                  bn_g0.reshape(1, G2), bn_b0.reshape(1, G2), W2_0,
                  b2_0.reshape(1, D))
  agg1 = _sc_segment_sum(h1, src2d, dst2d, zrows)
  out = _tc_layer2(h1, agg1, eps1.reshape(1, 1), W1_1, b1_1.reshape(1, G2),
                   bn_g1.reshape(1, G2), bn_b1.reshape(1, G2), W2_1,
                   b2_1.reshape(1, D), lin_W, lin_b.reshape(1, D))
  return out
